# baseline (speedup 1.0000x reference)
import functools

import jax
import jax.numpy as jnp
from jax import lax
from jax.experimental import pallas as pl
from jax.experimental.pallas import tpu as pltpu
from jax.experimental.pallas import tpu_sc as plsc

B = 16384
L = 20
D = 32
PK = 128

NC, NS = 2, 16
NW = NC * NS
SPW = B // NW
CH = 16
NCH = SPW // CH

_MESH = plsc.VectorSubcoreMesh(
    core_axis_name="c", subcore_axis_name="s", num_cores=NC, num_subcores=NS)


@functools.partial(
    pl.kernel,
    out_type=jax.ShapeDtypeStruct((B * 2 * D,), jnp.float32),
    mesh=_MESH,
    compiler_params=pltpu.CompilerParams(needs_layout_passes=False),
    scratch_types=[
        pltpu.VMEM((L * CH,), jnp.int32),
        pltpu.VMEM((L * CH,), jnp.int32),
        pltpu.VMEM((CH,), jnp.int32),
        pltpu.VMEM((CH,), jnp.int32),
        pltpu.VMEM((L * CH, PK), jnp.float32),
        pltpu.VMEM((L * CH, PK), jnp.float32),
        pltpu.VMEM((CH, PK), jnp.float32),
        pltpu.VMEM((CH, PK), jnp.float32),
        pltpu.VMEM((SPW * 2 * D,), jnp.float32),
        pltpu.SemaphoreType.DMA,
        pltpu.SemaphoreType.DMA,
        pltpu.SemaphoreType.DMA,
    ],
)
def _sc_kernel(title_hbm, text_hbm, tids_hbm, tb_hbm, out_hbm,
               ids0, ids1, tix0, tix1, rows0, rows1, trow0, trow1, outw,
               isem, gsem, tsem):
    wid = lax.axis_index("s") * NC + lax.axis_index("c")
    cid0 = wid * NCH

    def stage_idx(cidx, ib, xb):
        pltpu.async_copy(tb_hbm.at[pl.ds(cidx * (L * CH), L * CH)], ib, isem)
        pltpu.async_copy(tids_hbm.at[pl.ds(cidx * CH, CH)], xb, isem)

    def wait_idx(cidx, ib, xb):
        pltpu.make_async_copy(
            tb_hbm.at[pl.ds(cidx * (L * CH), L * CH)], ib, isem).wait()
        pltpu.make_async_copy(
            tids_hbm.at[pl.ds(cidx * CH, CH)], xb, isem).wait()

    def fire_gathers(ib, xb, rb, tb):
        for j in range(L):
            pltpu.async_copy(text_hbm.at[ib.at[pl.ds(j * CH, CH)]],
                             rb.at[pl.ds(j * CH, CH)], gsem)
        pltpu.async_copy(title_hbm.at[xb], tb, tsem)

    def wait_gathers(ib, xb, rb, tb):
        for j in range(L):
            pltpu.make_async_copy(text_hbm.at[ib.at[pl.ds(j * CH, CH)]],
                                  rb.at[pl.ds(j * CH, CH)], gsem).wait()
        pltpu.make_async_copy(title_hbm.at[xb], tb, tsem).wait()

    pltpu.sync_copy(tb_hbm.at[pl.ds(cid0 * (L * CH), L * CH)], ids0)
    pltpu.sync_copy(tids_hbm.at[pl.ds(cid0 * CH, CH)], tix0)
    fire_gathers(ids0, tix0, rows0, trow0)
    stage_idx(cid0 + 1, ids1, tix1)

    bufs = ((ids0, tix0, rows0, trow0), (ids1, tix1, rows1, trow1))

    def body(cc, _):
        for p in range(2):
            c = cc * 2 + p
            cidx = cid0 + c
            ib, xb, rb, tb = bufs[p]
            ibn, xbn, rbn, tbn = bufs[1 - p]

            @pl.when(c + 1 < NCH)
            def _():
                wait_idx(cidx + 1, ibn, xbn)
                fire_gathers(ibn, xbn, rbn, tbn)

            iot = lax.iota(jnp.int32, 16) * L
            cnt = jnp.zeros((16,), jnp.float32)
            for j in range(L):
                iv = plsc.load_gather(ib, [iot + j])
                cnt = cnt + jnp.where(iv != 0, 1.0, 0.0)
            rvec = 1.0 / jnp.maximum(cnt, 1.0)

            @pl.when(c + 2 < NCH)
            def _():
                stage_idx(cidx + 2, ib, xb)

            wait_gathers(ib, xb, rb, tb)

            for i2 in range(16):
                a0 = rb[i2 * L, pl.ds(0, 16)]
                a1 = rb[i2 * L, pl.ds(16, 16)]
                for j in range(1, L):
                    a0 = a0 + rb[i2 * L + j, pl.ds(0, 16)]
                    a1 = a1 + rb[i2 * L + j, pl.ds(16, 16)]
                r = rvec[i2]
                ob = pl.multiple_of(c * (CH * 2 * D) + i2 * 2 * D, 2 * D)
                outw[pl.ds(ob, 16)] = tb[i2, pl.ds(0, 16)]
                outw[pl.ds(ob + 16, 16)] = tb[i2, pl.ds(16, 16)]
                outw[pl.ds(ob + 32, 16)] = a0 * r
                outw[pl.ds(ob + 48, 16)] = a1 * r
        return 0

    lax.fori_loop(0, NCH // 2, body, 0)
    pltpu.sync_copy(outw, out_hbm.at[pl.ds(wid * (SPW * 2 * D), SPW * 2 * D)])


def kernel(title_table, text_table, title_ids, token_ids):
    text_z = text_table.at[0].set(0.0)
    text_p = jnp.pad(text_z, ((0, 0), (0, PK - D)))
    title_p = jnp.pad(title_table, ((0, 7), (0, PK - D)))
    tb = token_ids.reshape(-1)
    flat = _sc_kernel(title_p, text_p, title_ids, tb)
    return flat.reshape(B, 2 * D)
